# baseline (device time: 90114 ns/iter reference)
import jax
import jax.numpy as jnp
from jax import lax
from jax.experimental import pallas as pl
from jax.experimental.pallas import tpu as pltpu

N_GLOBAL = 4096
EPS = 1e-5
BM1 = 1536
BM2 = 1536


def _sumsq_allreduce(x):
    m, n = x.shape
    nb = m // BM1

    def body(x_ref, o_ref, acc, peer, send_sem, recv_sem):
        my_x = lax.axis_index("x")
        my_y = lax.axis_index("y")
        step = pl.program_id(0)

        @pl.when(step == 0)
        def _():
            barrier_sem = pltpu.get_barrier_semaphore()
            pl.semaphore_signal(
                barrier_sem,
                inc=1,
                device_id=(my_x, 1 - my_y),
                device_id_type=pl.DeviceIdType.MESH,
            )
            pl.semaphore_wait(barrier_sem, 1)

        xb = x_ref[...]
        acc[pl.ds(step * BM1, BM1), :] = jnp.sum(xb * xb, axis=1, keepdims=True)

        @pl.when(step == nb - 1)
        def _():
            rdma = pltpu.make_async_remote_copy(
                src_ref=acc,
                dst_ref=peer,
                send_sem=send_sem,
                recv_sem=recv_sem,
                device_id=(my_x, 1 - my_y),
                device_id_type=pl.DeviceIdType.MESH,
            )
            rdma.start()
            rdma.wait()
            o_ref[...] = acc[...] + peer[...]

    return pl.pallas_call(
        body,
        grid=(nb,),
        in_specs=[pl.BlockSpec((BM1, n), lambda i: (i, 0))],
        out_specs=pl.BlockSpec((m, 1), lambda i: (0, 0)),
        out_shape=jax.ShapeDtypeStruct((m, 1), jnp.float32),
        scratch_shapes=[
            pltpu.VMEM((m, 1), jnp.float32),
            pltpu.VMEM((m, 1), jnp.float32),
            pltpu.SemaphoreType.DMA,
            pltpu.SemaphoreType.DMA,
        ],
        compiler_params=pltpu.CompilerParams(
            collective_id=0, vmem_limit_bytes=100 * 1024 * 1024
        ),
    )(x)


def _normalize(x, gamma2, sumsq):
    m, n = x.shape
    nb = m // BM2

    def body(x_ref, g_ref, s_ref, o_ref):
        inv = lax.rsqrt(s_ref[...] * (1.0 / N_GLOBAL) + EPS)
        o_ref[...] = x_ref[...] * g_ref[...] * inv

    return pl.pallas_call(
        body,
        grid=(nb,),
        in_specs=[
            pl.BlockSpec((BM2, n), lambda i: (i, 0)),
            pl.BlockSpec((1, n), lambda i: (0, 0)),
            pl.BlockSpec((BM2, 1), lambda i: (i, 0)),
        ],
        out_specs=pl.BlockSpec((BM2, n), lambda i: (i, 0)),
        out_shape=jax.ShapeDtypeStruct((m, n), x.dtype),
        compiler_params=pltpu.CompilerParams(
            vmem_limit_bytes=100 * 1024 * 1024
        ),
    )(x, gamma2, sumsq)


def kernel(x, gamma):
    m, n = x.shape
    sumsq = _sumsq_allreduce(x)
    return _normalize(x, gamma.reshape(1, n), sumsq)


# device time: 55911 ns/iter; 1.6117x vs baseline; 1.6117x over previous
import jax
import jax.numpy as jnp
from jax import lax
from jax.experimental import pallas as pl
from jax.experimental.pallas import tpu as pltpu

N_GLOBAL = 4096
EPS = 1e-5
BM1 = 1536
BM2 = 1536


def _sumsq_allreduce(x):
    m, n = x.shape
    nb = m // BM1

    def body(x_ref, o_ref, acc, peer, send_sem, recv_sem):
        my_x = lax.axis_index("x")
        my_y = lax.axis_index("y")
        step = pl.program_id(0)

        @pl.when(step == 0)
        def _():
            barrier_sem = pltpu.get_barrier_semaphore()
            pl.semaphore_signal(
                barrier_sem,
                inc=1,
                device_id=(my_x, 1 - my_y),
                device_id_type=pl.DeviceIdType.MESH,
            )
            pl.semaphore_wait(barrier_sem, 1)

        xb = x_ref[...]
        p = jnp.sum(xb * xb, axis=1)
        acc[:, pl.ds(step * BM1, BM1)] = p[None, :]

        @pl.when(step == nb - 1)
        def _():
            rdma = pltpu.make_async_remote_copy(
                src_ref=acc,
                dst_ref=peer,
                send_sem=send_sem,
                recv_sem=recv_sem,
                device_id=(my_x, 1 - my_y),
                device_id_type=pl.DeviceIdType.MESH,
            )
            rdma.start()
            rdma.wait()
            o_ref[...] = acc[...] + peer[...]

    return pl.pallas_call(
        body,
        grid=(nb,),
        in_specs=[pl.BlockSpec((BM1, n), lambda i: (i, 0))],
        out_specs=pl.BlockSpec((1, m), lambda i: (0, 0)),
        out_shape=jax.ShapeDtypeStruct((1, m), jnp.float32),
        scratch_shapes=[
            pltpu.VMEM((1, m), jnp.float32),
            pltpu.VMEM((1, m), jnp.float32),
            pltpu.SemaphoreType.DMA,
            pltpu.SemaphoreType.DMA,
        ],
        compiler_params=pltpu.CompilerParams(
            collective_id=0, vmem_limit_bytes=100 * 1024 * 1024
        ),
    )(x)


def _normalize(x, gamma2, sumsq):
    m, n = x.shape
    nb = m // BM2

    def body(x_ref, g_ref, s_ref, o_ref):
        inv = lax.rsqrt(s_ref[...] * (1.0 / N_GLOBAL) + EPS)
        inv_col = jnp.transpose(inv, (1, 0))
        o_ref[...] = x_ref[...] * g_ref[...] * inv_col

    return pl.pallas_call(
        body,
        grid=(nb,),
        in_specs=[
            pl.BlockSpec((BM2, n), lambda i: (i, 0)),
            pl.BlockSpec((1, n), lambda i: (0, 0)),
            pl.BlockSpec((1, BM2), lambda i: (0, i)),
        ],
        out_specs=pl.BlockSpec((BM2, n), lambda i: (i, 0)),
        out_shape=jax.ShapeDtypeStruct((m, n), x.dtype),
        compiler_params=pltpu.CompilerParams(
            vmem_limit_bytes=100 * 1024 * 1024
        ),
    )(x, gamma2, sumsq)


def kernel(x, gamma):
    m, n = x.shape
    sumsq = _sumsq_allreduce(x)
    return _normalize(x, gamma.reshape(1, n), sumsq)


# device time: 54265 ns/iter; 1.6606x vs baseline; 1.0303x over previous
import jax
import jax.numpy as jnp
from jax import lax
from jax.experimental import pallas as pl
from jax.experimental.pallas import tpu as pltpu

N_GLOBAL = 4096
EPS = 1e-5
BM1 = 768
BM2 = 1536


def _sumsq_allreduce(x):
    m, n = x.shape
    nb = m // BM1

    def body(x_ref, o_ref, acc, peer, send_sem, recv_sem):
        my_x = lax.axis_index("x")
        my_y = lax.axis_index("y")
        step = pl.program_id(0)

        @pl.when(step == 0)
        def _():
            barrier_sem = pltpu.get_barrier_semaphore()
            pl.semaphore_signal(
                barrier_sem,
                inc=1,
                device_id=(my_x, 1 - my_y),
                device_id_type=pl.DeviceIdType.MESH,
            )
            pl.semaphore_wait(barrier_sem, 1)

        xb = x_ref[...]
        p = jnp.sum(xb * xb, axis=1)
        acc[:, pl.ds(step * BM1, BM1)] = p[None, :]

        @pl.when(step == nb - 1)
        def _():
            rdma = pltpu.make_async_remote_copy(
                src_ref=acc,
                dst_ref=peer,
                send_sem=send_sem,
                recv_sem=recv_sem,
                device_id=(my_x, 1 - my_y),
                device_id_type=pl.DeviceIdType.MESH,
            )
            rdma.start()
            rdma.wait()
            o_ref[...] = acc[...] + peer[...]

    return pl.pallas_call(
        body,
        grid=(nb,),
        in_specs=[pl.BlockSpec((BM1, n), lambda i: (i, 0))],
        out_specs=pl.BlockSpec((1, m), lambda i: (0, 0)),
        out_shape=jax.ShapeDtypeStruct((1, m), jnp.float32),
        scratch_shapes=[
            pltpu.VMEM((1, m), jnp.float32),
            pltpu.VMEM((1, m), jnp.float32),
            pltpu.SemaphoreType.DMA,
            pltpu.SemaphoreType.DMA,
        ],
        compiler_params=pltpu.CompilerParams(
            collective_id=0, vmem_limit_bytes=100 * 1024 * 1024
        ),
    )(x)


def _normalize(x, gamma2, sumsq):
    m, n = x.shape
    nb = m // BM2

    def body(x_ref, g_ref, s_ref, o_ref):
        inv = lax.rsqrt(s_ref[...] * (1.0 / N_GLOBAL) + EPS)
        inv_col = jnp.transpose(inv, (1, 0))
        o_ref[...] = x_ref[...] * g_ref[...] * inv_col

    return pl.pallas_call(
        body,
        grid=(nb,),
        in_specs=[
            pl.BlockSpec((BM2, n), lambda i: (i, 0)),
            pl.BlockSpec((1, n), lambda i: (0, 0)),
            pl.BlockSpec((1, BM2), lambda i: (0, i)),
        ],
        out_specs=pl.BlockSpec((BM2, n), lambda i: (i, 0)),
        out_shape=jax.ShapeDtypeStruct((m, n), x.dtype),
        compiler_params=pltpu.CompilerParams(
            vmem_limit_bytes=100 * 1024 * 1024
        ),
    )(x, gamma2, sumsq)


def kernel(x, gamma):
    m, n = x.shape
    sumsq = _sumsq_allreduce(x)
    return _normalize(x, gamma.reshape(1, n), sumsq)
